# baseline (speedup 1.0000x reference)
import functools

import jax
import jax.numpy as jnp
from jax import lax
from jax.experimental import pallas as pl
from jax.experimental.pallas import tpu as pltpu
from jax.experimental.pallas import tpu_sc as plsc

D, H, W = 160, 192, 224
HW = H * W
N = D * H * W
NC, NS, L = 2, 16, 16
NW = NC * NS
ZPW = D // NW
VPR = W // L
CHUNK = 112
NCH = W // CHUNK


def _warp_body(vol_hbm, trf_hbm, out_hbm, trf_v, idx_v, w_v, vals_v, out_v, sem):
    wid = lax.axis_index("s") * NC + lax.axis_index("c")
    z_base = wid * ZPW

    @pl.loop(0, ZPW)
    def _zloop(zi):
        z = z_base + zi

        @pl.loop(0, H)
        def _yloop(y):
            row = z * H + y
            pltpu.sync_copy(trf_hbm.at[pl.ds(row * (3 * W), 3 * W)], trf_v)
            zf = z.astype(jnp.float32)
            yf = y.astype(jnp.float32)
            iot = lax.iota(jnp.int32, L)
            iotf = iot.astype(jnp.float32)

            for v in range(VPR):
                tb = v * (3 * L) + iot * 3
                dzs = plsc.load_gather(trf_v, [tb])
                dys = plsc.load_gather(trf_v, [tb + 1])
                dxs = plsc.load_gather(trf_v, [tb + 2])
                fz = jnp.clip(zf + dzs, 0.0, float(D - 1))
                fy = jnp.clip(yf + dys, 0.0, float(H - 1))
                fx = jnp.clip(jnp.float32(v * L) + iotf + dxs, 0.0, float(W - 1))
                z0 = fz.astype(jnp.int32)
                y0 = fy.astype(jnp.int32)
                x0 = fx.astype(jnp.int32)
                wz1 = fz - z0.astype(jnp.float32)
                wy1 = fy - y0.astype(jnp.float32)
                wx1 = fx - x0.astype(jnp.float32)
                wz0 = 1.0 - wz1
                wy0 = 1.0 - wy1
                wx0 = 1.0 - wx1
                dzo = jnp.where(z0 < D - 1, HW, 0)
                dyo = jnp.where(y0 < H - 1, W, 0)
                dxo = jnp.where(x0 < W - 1, 1, 0)
                base = z0 * HW + y0 * W + x0
                c0 = base
                c1 = base + dxo
                c2 = base + dyo
                c3 = c2 + dxo
                c4 = base + dzo
                c5 = c4 + dxo
                c6 = c4 + dyo
                c7 = c6 + dxo
                a0 = wz0 * wy0
                a1 = wz0 * wy1
                a2 = wz1 * wy0
                a3 = wz1 * wy1
                ws = (a0 * wx0, a0 * wx1, a1 * wx0, a1 * wx1,
                      a2 * wx0, a2 * wx1, a3 * wx0, a3 * wx1)
                cs = (c0, c1, c2, c3, c4, c5, c6, c7)
                p = v * L
                ch, off = p // CHUNK, p % CHUNK
                for ci in range(8):
                    idx_v[ci, ch, pl.ds(off, L)] = cs[ci]
                    w_v[ci, pl.ds(p, L)] = ws[ci]

            descs = []
            for ci in range(8):
                for ch in range(NCH):
                    descs.append(pltpu.async_copy(
                        vol_hbm.at[idx_v.at[ci, ch]], vals_v.at[ci, ch], sem))
            for d in descs:
                d.wait()

            for v in range(VPR):
                p = v * L
                ch, off = p // CHUNK, p % CHUNK
                acc = w_v[0, pl.ds(p, L)] * vals_v[0, ch, pl.ds(off, L)]
                for ci in range(1, 8):
                    acc = acc + w_v[ci, pl.ds(p, L)] * vals_v[ci, ch, pl.ds(off, L)]
                out_v[pl.ds(p, L)] = acc

            pltpu.sync_copy(out_v, out_hbm.at[pl.ds(row * W, W)])


_warp = functools.partial(
    pl.kernel,
    out_type=jax.ShapeDtypeStruct((N,), jnp.float32),
    mesh=plsc.VectorSubcoreMesh(core_axis_name="c", subcore_axis_name="s",
                                num_cores=NC, num_subcores=NS),
    scratch_types=[
        pltpu.VMEM((3 * W,), jnp.float32),
        pltpu.VMEM((8, NCH, CHUNK), jnp.int32),
        pltpu.VMEM((8, W), jnp.float32),
        pltpu.VMEM((8, NCH, CHUNK), jnp.float32),
        pltpu.VMEM((W,), jnp.float32),
        pltpu.SemaphoreType.DMA,
    ],
    compiler_params=pltpu.CompilerParams(needs_layout_passes=False),
)(_warp_body)


def kernel(vol, trf):
    vol_flat = vol.reshape(N)
    trf_flat = trf.reshape(N * 3)
    out = _warp(vol_flat, trf_flat)
    return out.reshape(1, D, H, W, 1)


# speedup vs baseline: 1.0344x; 1.0344x over previous
import functools

import jax
import jax.numpy as jnp
from jax import lax
from jax.experimental import pallas as pl
from jax.experimental.pallas import tpu as pltpu
from jax.experimental.pallas import tpu_sc as plsc

D, H, W = 160, 192, 224
HW = H * W
N = D * H * W
NC, NS, L = 2, 16, 16
NW = NC * NS
RT = (D // NW) * H
CHUNK = 112
NCH = W // CHUNK
VPC = CHUNK // L


def _warp_body(vol_hbm, trf_hbm, out_hbm,
               trf_v0, trf_v1, idx_v0, idx_v1, w_v0, w_v1,
               vals_v0, vals_v1, out_v0, out_v1,
               trf_sem0, trf_sem1, g_sem0, g_sem1, o_sem0, o_sem1):
    trf_vs = (trf_v0, trf_v1)
    idx_vs = (idx_v0, idx_v1)
    w_vs = (w_v0, w_v1)
    vals_vs = (vals_v0, vals_v1)
    out_vs = (out_v0, out_v1)
    trf_sems = (trf_sem0, trf_sem1)
    g_sems = (g_sem0, g_sem1)
    o_sems = (o_sem0, o_sem1)
    wid = lax.axis_index("s") * NC + lax.axis_index("c")
    row0 = wid * RT
    iot = lax.iota(jnp.int32, L)
    iotf = iot.astype(jnp.float32)

    def fire_trf(b, s):
        pltpu.async_copy(
            trf_hbm.at[pl.ds((row0 + b) * (3 * W), 3 * W)],
            trf_vs[s], trf_sems[s])

    def wait_trf(s):
        pltpu.make_async_copy(
            trf_hbm.at[pl.ds(0, 3 * W)], trf_vs[s], trf_sems[s]).wait()

    def pass_a(b, s):
        grow = row0 + b
        z = grow // H
        y = grow - z * H
        zf = z.astype(jnp.float32)
        yf = y.astype(jnp.float32)
        for ch in range(NCH):
            for k in range(VPC):
                p = ch * CHUNK + k * L
                tb = p * 3 + iot * 3
                dzs = plsc.load_gather(trf_vs[s], [tb])
                dys = plsc.load_gather(trf_vs[s], [tb + 1])
                dxs = plsc.load_gather(trf_vs[s], [tb + 2])
                fz = jnp.clip(zf + dzs, 0.0, float(D - 1))
                fy = jnp.clip(yf + dys, 0.0, float(H - 1))
                fx = jnp.clip(jnp.float32(p) + iotf + dxs, 0.0, float(W - 1))
                z0 = fz.astype(jnp.int32)
                y0 = fy.astype(jnp.int32)
                x0 = fx.astype(jnp.int32)
                wz1 = fz - z0.astype(jnp.float32)
                wy1 = fy - y0.astype(jnp.float32)
                wx1 = fx - x0.astype(jnp.float32)
                wz0 = 1.0 - wz1
                wy0 = 1.0 - wy1
                wx0 = 1.0 - wx1
                dzo = jnp.where(z0 < D - 1, HW, 0)
                dyo = jnp.where(y0 < H - 1, W, 0)
                dxo = jnp.where(x0 < W - 1, 1, 0)
                base = z0 * HW + y0 * W + x0
                c0 = base
                c2 = base + dyo
                c4 = base + dzo
                c6 = c4 + dyo
                a0 = wz0 * wy0
                a1 = wz0 * wy1
                a2 = wz1 * wy0
                a3 = wz1 * wy1
                cs = (c0, c0 + dxo, c2, c2 + dxo, c4, c4 + dxo, c6, c6 + dxo)
                ws = (a0 * wx0, a0 * wx1, a1 * wx0, a1 * wx1,
                      a2 * wx0, a2 * wx1, a3 * wx0, a3 * wx1)
                for ci in range(8):
                    idx_vs[s][ci, ch, pl.ds(k * L, L)] = cs[ci]
                    w_vs[s][ci, pl.ds(p, L)] = ws[ci]

    def fire_gathers(s):
        for ci in range(8):
            for ch in range(NCH):
                pltpu.async_copy(vol_hbm.at[idx_vs[s].at[ci, ch]],
                                 vals_vs[s].at[ci, ch], g_sems[s])

    def wait_gathers(s):
        for _ in range(8 * NCH):
            pltpu.make_async_copy(vol_hbm.at[pl.ds(0, CHUNK)],
                                  vals_vs[s].at[0, 0], g_sems[s]).wait()

    def pass_b(b, s):
        for ch in range(NCH):
            for k in range(VPC):
                p = ch * CHUNK + k * L
                acc = (w_vs[s][0, pl.ds(p, L)]
                       * vals_vs[s][0, ch, pl.ds(k * L, L)])
                for ci in range(1, 8):
                    acc = acc + (w_vs[s][ci, pl.ds(p, L)]
                                 * vals_vs[s][ci, ch, pl.ds(k * L, L)])
                out_vs[s][pl.ds(p, L)] = acc
        pltpu.async_copy(out_vs[s],
                         out_hbm.at[pl.ds((row0 + b) * W, W)], o_sems[s])

    def wait_out(s):
        pltpu.make_async_copy(out_vs[s], out_hbm.at[pl.ds(0, W)],
                              o_sems[s]).wait()

    def phase(b, s, first, drain_out):
        wait_trf(s)
        fire_trf(jnp.minimum(b + 1, RT - 1), 1 - s)
        pass_a(b, s)
        if first:
            fire_gathers(s)
        else:
            wait_gathers(1 - s)
            fire_gathers(s)
            if drain_out is None:
                @pl.when(b >= 3)
                def _():
                    wait_out(1 - s)
            elif drain_out:
                wait_out(1 - s)
            pass_b(b - 1, 1 - s)

    fire_trf(jnp.int32(0), 0)
    phase(jnp.int32(0), 0, True, False)
    phase(jnp.int32(1), 1, False, False)

    @pl.loop(1, RT // 2)
    def _main(t):
        b = t * 2
        phase(b, 0, False, None)
        phase(b + 1, 1, False, None)

    wait_trf(0)
    wait_gathers(1)
    wait_out(1)
    pass_b(jnp.int32(RT - 1), 1)
    wait_out(0)
    wait_out(1)


_warp = functools.partial(
    pl.kernel,
    out_type=jax.ShapeDtypeStruct((N,), jnp.float32),
    mesh=plsc.VectorSubcoreMesh(core_axis_name="c", subcore_axis_name="s",
                                num_cores=NC, num_subcores=NS),
    scratch_types=[
        pltpu.VMEM((3 * W,), jnp.float32),
        pltpu.VMEM((3 * W,), jnp.float32),
        pltpu.VMEM((8, NCH, CHUNK), jnp.int32),
        pltpu.VMEM((8, NCH, CHUNK), jnp.int32),
        pltpu.VMEM((8, W), jnp.float32),
        pltpu.VMEM((8, W), jnp.float32),
        pltpu.VMEM((8, NCH, CHUNK), jnp.float32),
        pltpu.VMEM((8, NCH, CHUNK), jnp.float32),
        pltpu.VMEM((W,), jnp.float32),
        pltpu.VMEM((W,), jnp.float32),
        pltpu.SemaphoreType.DMA,
        pltpu.SemaphoreType.DMA,
        pltpu.SemaphoreType.DMA,
        pltpu.SemaphoreType.DMA,
        pltpu.SemaphoreType.DMA,
        pltpu.SemaphoreType.DMA,
    ],
    compiler_params=pltpu.CompilerParams(needs_layout_passes=False),
)(_warp_body)


def kernel(vol, trf):
    vol_flat = vol.reshape(N)
    trf_flat = trf.reshape(N * 3)
    out = _warp(vol_flat, trf_flat)
    return out.reshape(1, D, H, W, 1)


# speedup vs baseline: 7.7603x; 7.5024x over previous
import functools

import jax
import jax.numpy as jnp
from jax import lax
from jax.experimental import pallas as pl
from jax.experimental.pallas import tpu as pltpu
from jax.experimental.pallas import tpu_sc as plsc

D, H, W = 160, 192, 224
HW = H * W
N = D * H * W
NC, NS, L = 2, 16, 16
NW = NC * NS
RT = (D // NW) * H
CHUNK = 112
NCH = W // CHUNK
VPC = CHUNK // L


def _warp_body(vol_hbm, dz_hbm, dy_hbm, dx_hbm, out_hbm,
               trf_v0, trf_v1, idx_v0, idx_v1, w_v0, w_v1,
               vals_v0, vals_v1, out_v0, out_v1,
               trf_sem0, trf_sem1, g_sem0, g_sem1, o_sem0, o_sem1):
    trf_vs = (trf_v0, trf_v1)
    idx_vs = (idx_v0, idx_v1)
    w_vs = (w_v0, w_v1)
    vals_vs = (vals_v0, vals_v1)
    out_vs = (out_v0, out_v1)
    trf_sems = (trf_sem0, trf_sem1)
    g_sems = (g_sem0, g_sem1)
    o_sems = (o_sem0, o_sem1)
    wid = lax.axis_index("s") * NC + lax.axis_index("c")
    row0 = wid * RT
    iot = lax.iota(jnp.int32, L)
    iotf = iot.astype(jnp.float32)

    def fire_trf(b, s):
        off = (row0 + b) * W
        pltpu.async_copy(dz_hbm.at[pl.ds(off, W)],
                         trf_vs[s].at[pl.ds(0, W)], trf_sems[s])
        pltpu.async_copy(dy_hbm.at[pl.ds(off, W)],
                         trf_vs[s].at[pl.ds(W, W)], trf_sems[s])
        pltpu.async_copy(dx_hbm.at[pl.ds(off, W)],
                         trf_vs[s].at[pl.ds(2 * W, W)], trf_sems[s])

    def wait_trf(s):
        for i in range(3):
            pltpu.make_async_copy(dz_hbm.at[pl.ds(0, W)],
                                  trf_vs[s].at[pl.ds(0, W)],
                                  trf_sems[s]).wait()

    def pass_a(b, s):
        grow = row0 + b
        z = grow // H
        y = grow - z * H
        zf = z.astype(jnp.float32)
        yf = y.astype(jnp.float32)
        for ch in range(NCH):
            for k in range(VPC):
                p = ch * CHUNK + k * L
                dzs = trf_vs[s][pl.ds(p, L)]
                dys = trf_vs[s][pl.ds(W + p, L)]
                dxs = trf_vs[s][pl.ds(2 * W + p, L)]
                fz = jnp.clip(zf + dzs, 0.0, float(D - 1))
                fy = jnp.clip(yf + dys, 0.0, float(H - 1))
                fx = jnp.clip(jnp.float32(p) + iotf + dxs, 0.0, float(W - 1))
                z0 = fz.astype(jnp.int32)
                y0 = fy.astype(jnp.int32)
                x0 = fx.astype(jnp.int32)
                wz1 = fz - z0.astype(jnp.float32)
                wy1 = fy - y0.astype(jnp.float32)
                wx1 = fx - x0.astype(jnp.float32)
                wz0 = 1.0 - wz1
                wy0 = 1.0 - wy1
                wx0 = 1.0 - wx1
                dzo = jnp.where(z0 < D - 1, HW, 0)
                dyo = jnp.where(y0 < H - 1, W, 0)
                dxo = jnp.where(x0 < W - 1, 1, 0)
                base = z0 * HW + y0 * W + x0
                c0 = base
                c2 = base + dyo
                c4 = base + dzo
                c6 = c4 + dyo
                a0 = wz0 * wy0
                a1 = wz0 * wy1
                a2 = wz1 * wy0
                a3 = wz1 * wy1
                cs = (c0, c0 + dxo, c2, c2 + dxo, c4, c4 + dxo, c6, c6 + dxo)
                ws = (a0 * wx0, a0 * wx1, a1 * wx0, a1 * wx1,
                      a2 * wx0, a2 * wx1, a3 * wx0, a3 * wx1)
                for ci in range(8):
                    idx_vs[s][ci, ch, pl.ds(k * L, L)] = cs[ci]
                    w_vs[s][ci, pl.ds(p, L)] = ws[ci]

    def fire_gathers(s):
        for ci in range(8):
            for ch in range(NCH):
                pltpu.async_copy(vol_hbm.at[idx_vs[s].at[ci, ch]],
                                 vals_vs[s].at[ci, ch], g_sems[s])

    def wait_gathers(s):
        for _ in range(8 * NCH):
            pltpu.make_async_copy(vol_hbm.at[pl.ds(0, CHUNK)],
                                  vals_vs[s].at[0, 0], g_sems[s]).wait()

    def pass_b(b, s):
        for ch in range(NCH):
            for k in range(VPC):
                p = ch * CHUNK + k * L
                acc = (w_vs[s][0, pl.ds(p, L)]
                       * vals_vs[s][0, ch, pl.ds(k * L, L)])
                for ci in range(1, 8):
                    acc = acc + (w_vs[s][ci, pl.ds(p, L)]
                                 * vals_vs[s][ci, ch, pl.ds(k * L, L)])
                out_vs[s][pl.ds(p, L)] = acc
        pltpu.async_copy(out_vs[s],
                         out_hbm.at[pl.ds((row0 + b) * W, W)], o_sems[s])

    def wait_out(s):
        pltpu.make_async_copy(out_vs[s], out_hbm.at[pl.ds(0, W)],
                              o_sems[s]).wait()

    def phase(b, s, first, drain_out):
        wait_trf(s)
        fire_trf(jnp.minimum(b + 1, RT - 1), 1 - s)
        pass_a(b, s)
        if first:
            fire_gathers(s)
        else:
            wait_gathers(1 - s)
            fire_gathers(s)
            if drain_out is None:
                @pl.when(b >= 3)
                def _():
                    wait_out(1 - s)
            elif drain_out:
                wait_out(1 - s)
            pass_b(b - 1, 1 - s)

    fire_trf(jnp.int32(0), 0)
    phase(jnp.int32(0), 0, True, False)
    phase(jnp.int32(1), 1, False, False)

    @pl.loop(1, RT // 2)
    def _main(t):
        b = t * 2
        phase(b, 0, False, None)
        phase(b + 1, 1, False, None)

    wait_trf(0)
    wait_gathers(1)
    wait_out(1)
    pass_b(jnp.int32(RT - 1), 1)
    wait_out(0)
    wait_out(1)


_warp = functools.partial(
    pl.kernel,
    out_type=jax.ShapeDtypeStruct((N,), jnp.float32),
    mesh=plsc.VectorSubcoreMesh(core_axis_name="c", subcore_axis_name="s",
                                num_cores=NC, num_subcores=NS),
    scratch_types=[
        pltpu.VMEM((3 * W,), jnp.float32),
        pltpu.VMEM((3 * W,), jnp.float32),
        pltpu.VMEM((8, NCH, CHUNK), jnp.int32),
        pltpu.VMEM((8, NCH, CHUNK), jnp.int32),
        pltpu.VMEM((8, W), jnp.float32),
        pltpu.VMEM((8, W), jnp.float32),
        pltpu.VMEM((8, NCH, CHUNK), jnp.float32),
        pltpu.VMEM((8, NCH, CHUNK), jnp.float32),
        pltpu.VMEM((W,), jnp.float32),
        pltpu.VMEM((W,), jnp.float32),
        pltpu.SemaphoreType.DMA,
        pltpu.SemaphoreType.DMA,
        pltpu.SemaphoreType.DMA,
        pltpu.SemaphoreType.DMA,
        pltpu.SemaphoreType.DMA,
        pltpu.SemaphoreType.DMA,
    ],
    compiler_params=pltpu.CompilerParams(needs_layout_passes=False),
)(_warp_body)


def kernel(vol, trf):
    dz = trf[0, :, :, :, 0].reshape(N)
    dy = trf[0, :, :, :, 1].reshape(N)
    dx = trf[0, :, :, :, 2].reshape(N)
    vol_flat = vol.reshape(N)
    out = _warp(vol_flat, dz, dy, dx)
    return out.reshape(1, D, H, W, 1)


# speedup vs baseline: 11.3785x; 1.4662x over previous
import functools

import jax
import jax.numpy as jnp
from jax import lax
from jax.experimental import pallas as pl
from jax.experimental.pallas import tpu as pltpu
from jax.experimental.pallas import tpu_sc as plsc

D, H, W = 160, 192, 224
HW = H * W
N = D * H * W
NC, NS, L = 2, 16, 16
NW = NC * NS
RT = (D // NW) * H
CHUNK = 112
NCH = W // CHUNK
VPC = CHUNK // L


def _warp_body(vol_hbm, dz_hbm, dy_hbm, dx_hbm, out_hbm,
               trf_v0, trf_v1, idx_v0, idx_v1, o_v0, o_v1, w_v0, w_v1,
               vals_v0, vals_v1, out_v0, out_v1,
               trf_sem0, trf_sem1, g_sem0, g_sem1, o_sem0, o_sem1):
    trf_vs = (trf_v0, trf_v1)
    idx_vs = (idx_v0, idx_v1)
    o_vs = (o_v0, o_v1)
    w_vs = (w_v0, w_v1)
    vals_vs = (vals_v0, vals_v1)
    out_vs = (out_v0, out_v1)
    trf_sems = (trf_sem0, trf_sem1)
    g_sems = (g_sem0, g_sem1)
    o_sems = (o_sem0, o_sem1)
    wid = lax.axis_index("s") * NC + lax.axis_index("c")
    row0 = wid * RT
    iot = lax.iota(jnp.int32, L)
    iotf = iot.astype(jnp.float32)

    def fire_trf(b, s):
        off = (row0 + b) * W
        pltpu.async_copy(dz_hbm.at[pl.ds(off, W)],
                         trf_vs[s].at[pl.ds(0, W)], trf_sems[s])
        pltpu.async_copy(dy_hbm.at[pl.ds(off, W)],
                         trf_vs[s].at[pl.ds(W, W)], trf_sems[s])
        pltpu.async_copy(dx_hbm.at[pl.ds(off, W)],
                         trf_vs[s].at[pl.ds(2 * W, W)], trf_sems[s])

    def wait_trf(s):
        for i in range(3):
            pltpu.make_async_copy(dz_hbm.at[pl.ds(0, W)],
                                  trf_vs[s].at[pl.ds(0, W)],
                                  trf_sems[s]).wait()

    def pass_a(b, s):
        grow = row0 + b
        z = grow // H
        y = grow - z * H
        zf = z.astype(jnp.float32)
        yf = y.astype(jnp.float32)
        for ch in range(NCH):
            for k in range(VPC):
                p = ch * CHUNK + k * L
                dzs = trf_vs[s][pl.ds(p, L)]
                dys = trf_vs[s][pl.ds(W + p, L)]
                dxs = trf_vs[s][pl.ds(2 * W + p, L)]
                fz = jnp.clip(zf + dzs, 0.0, float(D - 1))
                fy = jnp.clip(yf + dys, 0.0, float(H - 1))
                fx = jnp.clip(jnp.float32(p) + iotf + dxs, 0.0, float(W - 1))
                z0 = fz.astype(jnp.int32)
                y0 = fy.astype(jnp.int32)
                x0 = fx.astype(jnp.int32)
                wz1 = fz - z0.astype(jnp.float32)
                wy1 = fy - y0.astype(jnp.float32)
                wx1 = fx - x0.astype(jnp.float32)
                wz0 = 1.0 - wz1
                wy0 = 1.0 - wy1
                wx0 = 1.0 - wx1
                dzo = jnp.where(z0 < D - 1, HW, 0)
                dyo = jnp.where(y0 < H - 1, W, 0)
                base = z0 * HW + y0 * W + x0
                a0 = wz0 * wy0
                a1 = wz0 * wy1
                a2 = wz1 * wy0
                a3 = wz1 * wy1
                cs = (base, base + dyo, base + dzo, base + dzo + dyo)
                ws = (a0 * wx0, a0 * wx1, a1 * wx0, a1 * wx1,
                      a2 * wx0, a2 * wx1, a3 * wx0, a3 * wx1)
                for pi in range(4):
                    idx_vs[s][pi, ch, pl.ds(k * L, L)] = cs[pi] >> 3
                    o_vs[s][pi, pl.ds(p, L)] = jnp.bitwise_and(cs[pi], 7)
                for ci in range(8):
                    w_vs[s][ci, pl.ds(p, L)] = ws[ci]

    def fire_gathers(s):
        for pi in range(4):
            for ch in range(NCH):
                pltpu.async_copy(vol_hbm.at[idx_vs[s].at[pi, ch]],
                                 vals_vs[s].at[pi, ch], g_sems[s])

    def wait_gathers(s):
        for _ in range(4 * NCH):
            pltpu.make_async_copy(vol_hbm.at[pl.ds(0, CHUNK), :],
                                  vals_vs[s].at[0, 0], g_sems[s]).wait()

    def pass_b(b, s):
        for ch in range(NCH):
            chv = jnp.full((L,), ch, jnp.int32)
            for k in range(VPC):
                p = ch * CHUNK + k * L
                jv = iot + (k * L)
                acc = None
                for pi in range(4):
                    piv = jnp.full((L,), pi, jnp.int32)
                    ov = o_vs[s][pi, pl.ds(p, L)]
                    v0 = plsc.load_gather(vals_vs[s], [piv, chv, jv, ov])
                    v1 = plsc.load_gather(vals_vs[s], [piv, chv, jv, ov + 1])
                    term = (w_vs[s][2 * pi, pl.ds(p, L)] * v0
                            + w_vs[s][2 * pi + 1, pl.ds(p, L)] * v1)
                    acc = term if acc is None else acc + term
                out_vs[s][pl.ds(p, L)] = acc
        pltpu.async_copy(out_vs[s],
                         out_hbm.at[pl.ds((row0 + b) * W, W)], o_sems[s])

    def wait_out(s):
        pltpu.make_async_copy(out_vs[s], out_hbm.at[pl.ds(0, W)],
                              o_sems[s]).wait()

    def phase(b, s, first, drain_out):
        wait_trf(s)
        fire_trf(jnp.minimum(b + 1, RT - 1), 1 - s)
        pass_a(b, s)
        if first:
            fire_gathers(s)
        else:
            wait_gathers(1 - s)
            fire_gathers(s)
            if drain_out is None:
                @pl.when(b >= 3)
                def _():
                    wait_out(1 - s)
            elif drain_out:
                wait_out(1 - s)
            pass_b(b - 1, 1 - s)

    fire_trf(jnp.int32(0), 0)
    phase(jnp.int32(0), 0, True, False)
    phase(jnp.int32(1), 1, False, False)

    @pl.loop(1, RT // 2)
    def _main(t):
        b = t * 2
        phase(b, 0, False, None)
        phase(b + 1, 1, False, None)

    wait_trf(0)
    wait_gathers(1)
    wait_out(1)
    pass_b(jnp.int32(RT - 1), 1)
    wait_out(0)
    wait_out(1)


_warp = functools.partial(
    pl.kernel,
    out_type=jax.ShapeDtypeStruct((N,), jnp.float32),
    mesh=plsc.VectorSubcoreMesh(core_axis_name="c", subcore_axis_name="s",
                                num_cores=NC, num_subcores=NS),
    scratch_types=[
        pltpu.VMEM((3 * W,), jnp.float32),
        pltpu.VMEM((3 * W,), jnp.float32),
        pltpu.VMEM((4, NCH, CHUNK), jnp.int32),
        pltpu.VMEM((4, NCH, CHUNK), jnp.int32),
        pltpu.VMEM((4, W), jnp.int32),
        pltpu.VMEM((4, W), jnp.int32),
        pltpu.VMEM((8, W), jnp.float32),
        pltpu.VMEM((8, W), jnp.float32),
        pltpu.VMEM((4, NCH, CHUNK, 16), jnp.float32),
        pltpu.VMEM((4, NCH, CHUNK, 16), jnp.float32),
        pltpu.VMEM((W,), jnp.float32),
        pltpu.VMEM((W,), jnp.float32),
        pltpu.SemaphoreType.DMA,
        pltpu.SemaphoreType.DMA,
        pltpu.SemaphoreType.DMA,
        pltpu.SemaphoreType.DMA,
        pltpu.SemaphoreType.DMA,
        pltpu.SemaphoreType.DMA,
    ],
    compiler_params=pltpu.CompilerParams(needs_layout_passes=False, use_tc_tiling_on_sc=False),
)(_warp_body)


NE = N // NW
BK = NE // 64


def _pair_body(vol_hbm, out_hbm, in_v, out_v):
    wid = lax.axis_index("s") * NC + lax.axis_index("c")
    base = wid * NE
    iot = lax.iota(jnp.int32, L)

    @pl.loop(0, NE // BK)
    def _blk(bi):
        off = base + bi * BK
        pltpu.sync_copy(vol_hbm.at[pl.ds(off, BK)], in_v.at[pl.ds(0, BK)])
        tail = jnp.minimum(off + BK, N - L)
        pltpu.sync_copy(vol_hbm.at[pl.ds(tail, L)], in_v.at[pl.ds(BK, L)])

        @pl.loop(0, BK // 8)
        def _vec(j):
            v = plsc.load_gather(in_v, [iot + j * 8])
            plsc.store_scatter(out_v, [iot + j * L], v)
        pltpu.sync_copy(out_v, out_hbm.at[pl.ds(2 * off, 2 * BK)])


_pairs = functools.partial(
    pl.kernel,
    out_type=jax.ShapeDtypeStruct((2 * N,), jnp.float32),
    mesh=plsc.VectorSubcoreMesh(core_axis_name="c", subcore_axis_name="s",
                                num_cores=NC, num_subcores=NS),
    scratch_types=[
        pltpu.VMEM((BK + L,), jnp.float32),
        pltpu.VMEM((2 * BK,), jnp.float32),
    ],
    compiler_params=pltpu.CompilerParams(needs_layout_passes=False,
                                         use_tc_tiling_on_sc=False),
)(_pair_body)


def kernel(vol, trf):
    dz = trf[0, :, :, :, 0].reshape(N)
    dy = trf[0, :, :, :, 1].reshape(N)
    dx = trf[0, :, :, :, 2].reshape(N)
    vol_flat = vol.reshape(N)
    vol_pairs = _pairs(vol_flat).reshape(N // 8, 16)
    out = _warp(vol_pairs, dz, dy, dx)
    return out.reshape(1, D, H, W, 1)
